# baseline (speedup 1.0000x reference)
import jax
import jax.numpy as jnp
from jax import lax
from jax.experimental import pallas as pl
from jax.experimental.pallas import tpu as pltpu
from jax.experimental.pallas import tpu_sc as plsc

_L = 16
_BLK = 128
_NW = 32


def _make_proj_body(n, n_pad, e_blk, eb_pad):
    def body(x_ref, wl_ref, wr_ref, ei_ref, o1_ref, o2_ref, src_ref, dst_ref,
             zc_ref, id_ref):
        x = x_ref[...]
        zt = jnp.zeros((n_pad - n, _L), jnp.float32)
        o1_ref[pl.ds(0, n), :] = jnp.dot(x, wl_ref[...],
                                         preferred_element_type=jnp.float32)
        o1_ref[pl.ds(n, n_pad - n), :] = zt
        o2_ref[pl.ds(0, n), :] = jnp.dot(x, wr_ref[...],
                                         preferred_element_type=jnp.float32)
        o2_ref[pl.ds(n, n_pad - n), :] = zt
        src_ref[pl.ds(0, e_blk), :] = ei_ref[0].reshape(e_blk, _BLK)
        src_ref[pl.ds(e_blk, eb_pad - e_blk), :] = jnp.zeros(
            (eb_pad - e_blk, _BLK), jnp.int32)
        dst_ref[pl.ds(0, e_blk), :] = ei_ref[1].reshape(e_blk, _BLK)
        dst_ref[pl.ds(e_blk, eb_pad - e_blk), :] = jnp.full(
            (eb_pad - e_blk, _BLK), n, jnp.int32)
        zc_ref[...] = jnp.zeros((n_pad, _L), jnp.float32)
        id_ref[...] = (_BLK * lax.broadcasted_iota(jnp.int32,
                                                   (n_pad // 16 // _BLK, _BLK),
                                                   0)
                       + lax.broadcasted_iota(jnp.int32,
                                              (n_pad // 16 // _BLK, _BLK), 1))
    return body


def _make_final_body(n):
    def body(agg_ref, deg_ref, h_ref, wl_ref, wr_ref, b2_ref, o_ref):
        agg = agg_ref[0, pl.ds(0, n)] + agg_ref[1, pl.ds(0, n)]
        deg = jnp.maximum(deg_ref[0, pl.ds(0, n // _L)]
                          + deg_ref[1, pl.ds(0, n // _L)], 1.0)
        mean3 = agg.reshape(n // _L, _L, _L) / deg[:, :, None]
        mean2 = mean3.reshape(n, _L)
        o_ref[...] = (jnp.dot(mean2, wl_ref[...],
                              preferred_element_type=jnp.float32)
                      + jnp.dot(h_ref[0, pl.ds(0, n)], wr_ref[...],
                                preferred_element_type=jnp.float32)
                      + b2_ref[...])
    return body


def _make_segsum(n_pad, nblk_tile, mode):
    mesh = plsc.VectorSubcoreMesh(core_axis_name="c", subcore_axis_name="s")
    rps = n_pad // 16
    nrd = n_pad // 16
    nrd_blk = nrd // _BLK
    drps = nrd // 16
    grp = 8
    grows = grp * _BLK
    assert nblk_tile % (2 * grp) == 0 and nrd % _BLK == 0 and drps % 8 == 0
    ngrp = nblk_tile // grp

    out_type = [jax.ShapeDtypeStruct((2, n_pad, _L), jnp.float32)]
    scratch = [
        pltpu.VMEM((nblk_tile, _BLK), jnp.int32),
        pltpu.VMEM((nblk_tile, _BLK), jnp.int32),
        pltpu.VMEM((2, grows, _L), jnp.float32),
        pltpu.VMEM_SHARED((n_pad, _L), jnp.float32),
        pltpu.SemaphoreType.DMA((2,)),
        pltpu.SemaphoreType.DMA((2,)),
    ]
    if mode == "deg":
        out_type.append(jax.ShapeDtypeStruct((2, nrd, _L), jnp.float32))
        scratch += [
            pltpu.VMEM((nrd_blk, _BLK), jnp.int32),
            pltpu.VMEM((nrd, _L), jnp.float32),
            pltpu.VMEM_SHARED((nrd, _L), jnp.float32),
            pltpu.SemaphoreType.DMA,
        ]
    else:
        out_type.append(jax.ShapeDtypeStruct((2, n_pad, _L), jnp.float32))
        scratch += [
            pltpu.VMEM((rps, _L), jnp.float32),
            pltpu.VMEM((rps, _L), jnp.float32),
            pltpu.VMEM((rps, _L), jnp.float32),
            pltpu.VMEM((2, drps, _L), jnp.float32),
            pltpu.VMEM((1, _L), jnp.float32),
            pltpu.VMEM((rps, _L), jnp.float32),
        ]

    def body(y_hbm, src_hbm, dst_hbm, zc_hbm, *rest):
        if mode == "deg":
            (id_hbm, out_hbm, degout_hbm, src_v, dst_v, rows_v, acc_sh,
             sem_g, sem_s, id_v, deg_v, dacc_sh, sem_d) = rest
        else:
            (a_hbm, dg_hbm, b1_hbm, out_hbm, h_hbm, src_v, dst_v, rows_v,
             acc_sh, sem_g, sem_s, ap_v, bp_v, z1_v, dg_v, b1_v,
             h_v) = rest

        c = lax.axis_index("c")
        s = lax.axis_index("s")
        wid = s * 2 + c

        if mode == "deg":
            gsrc = y_hbm
        else:
            gsrc = h_hbm.at[c]

        def issue_gathers(k, buf):
            for b in range(grp):
                pltpu.async_copy(
                    gsrc.at[src_v.at[k * grp + b]],
                    rows_v.at[buf, pl.ds(b * _BLK, _BLK)], sem_g.at[buf])

        def drain(sem):
            pltpu.make_async_copy(
                y_hbm.at[pl.ds(0, grows)], rows_v.at[0], sem).wait()

        start = wid * nblk_tile
        my_rows = pl.ds(s * rps, rps)
        pltpu.sync_copy(src_hbm.at[pl.ds(start, nblk_tile)], src_v)
        pltpu.sync_copy(dst_hbm.at[pl.ds(start, nblk_tile)], dst_v)
        pltpu.sync_copy(zc_hbm.at[my_rows], acc_sh.at[my_rows])
        if mode == "deg":
            my_drows = pl.ds(s * drps, drps)
            pltpu.sync_copy(id_hbm, id_v)
            pltpu.sync_copy(zc_hbm.at[pl.ds(0, nrd)], deg_v)
            pltpu.sync_copy(zc_hbm.at[pl.ds(0, drps)], dacc_sh.at[my_drows])
        else:
            my_drows = pl.ds(s * drps, drps)
            pltpu.sync_copy(a_hbm.at[0, my_rows], ap_v)
            pltpu.sync_copy(a_hbm.at[1, my_rows], bp_v)
            pltpu.sync_copy(y_hbm.at[my_rows], z1_v)
            pltpu.sync_copy(dg_hbm.at[0, my_drows], dg_v.at[0])
            pltpu.sync_copy(dg_hbm.at[1, my_drows], dg_v.at[1])
            pltpu.sync_copy(b1_hbm, b1_v)
            b1vec = b1_v[0, :]

            @pl.loop(0, drps)
            def _(r):
                deg16 = jnp.maximum(dg_v[0, r, :] + dg_v[1, r, :], 1.0)
                for i in range(_L):
                    row = r * _L + i
                    dsc = jnp.sum(jnp.where(
                        lax.iota(jnp.int32, _L) == i, deg16, 0.0))
                    dvec = jnp.full((_L,), dsc, jnp.float32)
                    mean = (ap_v[row, :] + bp_v[row, :]) / dvec
                    h_v[row, :] = jnp.maximum(
                        mean + b1vec + z1_v[row, :], 0.0)

            pltpu.sync_copy(h_v, h_hbm.at[c, my_rows])

        plsc.subcore_barrier()
        issue_gathers(0, 0)
        issue_gathers(1, 1)

        ones16 = jnp.ones((_L,), jnp.float32)

        @pl.loop(0, ngrp // 2)
        def _(g2):
            for buf in range(2):
                k = g2 * 2 + buf
                drain(sem_g.at[buf])
                for b in range(grp):
                    j = k * grp + b
                    pltpu.async_copy(
                        rows_v.at[buf, pl.ds(b * _BLK, _BLK)],
                        acc_sh.at[dst_v.at[j]], sem_s.at[buf], add=True)
                if mode == "deg":
                    for b in range(grp):
                        j = k * grp + b
                        for q in range(_BLK // _L):
                            d16 = dst_v.at[j][pl.ds(q * _L, _L)]
                            plsc.addupdate_scatter(
                                deg_v,
                                [lax.shift_right_logical(d16, 4),
                                 lax.bitwise_and(d16, 15)],
                                ones16)
                drain(sem_s.at[buf])

                @pl.when(k + 2 < ngrp)
                def _():
                    issue_gathers(k + 2, buf)

        if mode == "deg":
            for b in range(nrd_blk):
                pltpu.async_copy(deg_v.at[pl.ds(b * _BLK, _BLK)],
                                 dacc_sh.at[id_v.at[b]], sem_d, add=True)
            pltpu.make_async_copy(
                y_hbm.at[pl.ds(0, nrd)], deg_v, sem_d).wait()

        plsc.subcore_barrier()

        pltpu.sync_copy(acc_sh.at[my_rows], out_hbm.at[c, my_rows])
        if mode == "deg":
            my_drows = pl.ds(s * drps, drps)
            pltpu.sync_copy(dacc_sh.at[my_drows], degout_hbm.at[c, my_drows])

    return pl.kernel(
        body,
        out_type=tuple(out_type),
        mesh=mesh,
        scratch_types=scratch,
        compiler_params=pltpu.CompilerParams(use_tc_tiling_on_sc=False,
                                             needs_layout_passes=False),
    )


@jax.jit
def kernel(x, edge_index, W1_l, b1, W1_r, W2_l, b2, W2_r):
    n, d = x.shape
    h_dim = W1_l.shape[1]
    e = edge_index.shape[1]
    assert h_dim == _L and W2_l.shape[1] == _L

    blk_per_tile = -(-e // (_BLK * _NW))
    blk_per_tile = -(-blk_per_tile // 16) * 16
    e_pad = blk_per_tile * _BLK * _NW
    n_pad = -(-(n + 1) // 2048) * 2048
    assert e % _BLK == 0
    e_blk, eb_pad = e // _BLK, e_pad // _BLK

    segsum_deg = _make_segsum(n_pad, blk_per_tile, mode="deg")
    segsum_h = _make_segsum(n_pad, blk_per_tile, mode="h")

    y1p, z1p, src, dst, zc, idx_id = pl.pallas_call(
        _make_proj_body(n, n_pad, e_blk, eb_pad),
        out_shape=[jax.ShapeDtypeStruct((n_pad, _L), jnp.float32),
                   jax.ShapeDtypeStruct((n_pad, _L), jnp.float32),
                   jax.ShapeDtypeStruct((eb_pad, _BLK), jnp.int32),
                   jax.ShapeDtypeStruct((eb_pad, _BLK), jnp.int32),
                   jax.ShapeDtypeStruct((n_pad, _L), jnp.float32),
                   jax.ShapeDtypeStruct((n_pad // 16 // _BLK, _BLK),
                                        jnp.int32)],
    )(x, W1_l, W1_r, edge_index)

    agg1p, degp = segsum_deg(y1p, src, dst, zc, idx_id)

    agg2p, h = segsum_h(z1p, src, dst, zc, agg1p, degp, b1.reshape(1, _L))

    out = pl.pallas_call(
        _make_final_body(n),
        out_shape=jax.ShapeDtypeStruct((n, _L), jnp.float32),
    )(agg2p, degp, h, W2_l, W2_r, b2.reshape(1, _L))
    return out


# speedup vs baseline: 1.6530x; 1.6530x over previous
import jax
import jax.numpy as jnp
from jax import lax
from jax.experimental import pallas as pl
from jax.experimental.pallas import tpu as pltpu
from jax.experimental.pallas import tpu_sc as plsc

_L = 16
_BLK = 128
_NW = 32


def _make_proj_body(n, n_pad, e_blk, eb_pad):
    def body(x_ref, wl_ref, wr_ref, ei_ref, o1_ref, o2_ref, src_ref, dst_ref,
             zc_ref, id_ref):
        x = x_ref[...]
        zt = jnp.zeros((n_pad - n, _L), jnp.float32)
        o1_ref[pl.ds(0, n), :] = jnp.dot(x, wl_ref[...],
                                         preferred_element_type=jnp.float32)
        o1_ref[pl.ds(n, n_pad - n), :] = zt
        o2_ref[pl.ds(0, n), :] = jnp.dot(x, wr_ref[...],
                                         preferred_element_type=jnp.float32)
        o2_ref[pl.ds(n, n_pad - n), :] = zt
        src_ref[pl.ds(0, e_blk), :] = ei_ref[0].reshape(e_blk, _BLK)
        src_ref[pl.ds(e_blk, eb_pad - e_blk), :] = jnp.zeros(
            (eb_pad - e_blk, _BLK), jnp.int32)
        dst_ref[pl.ds(0, e_blk), :] = ei_ref[1].reshape(e_blk, _BLK)
        dst_ref[pl.ds(e_blk, eb_pad - e_blk), :] = jnp.full(
            (eb_pad - e_blk, _BLK), n, jnp.int32)
        zc_ref[...] = jnp.zeros((n_pad, _L), jnp.float32)
        id_ref[...] = (_BLK * lax.broadcasted_iota(jnp.int32,
                                                   (n_pad // 16 // _BLK, _BLK),
                                                   0)
                       + lax.broadcasted_iota(jnp.int32,
                                              (n_pad // 16 // _BLK, _BLK), 1))
    return body


def _make_final_body(n):
    def body(agg_ref, deg_ref, h_ref, wl_ref, wr_ref, b2_ref, o_ref):
        agg = agg_ref[0, pl.ds(0, n)] + agg_ref[1, pl.ds(0, n)]
        deg = jnp.maximum(deg_ref[0, pl.ds(0, n // _L)]
                          + deg_ref[1, pl.ds(0, n // _L)], 1.0)
        mean3 = agg.reshape(n // _L, _L, _L) / deg[:, :, None]
        mean2 = mean3.reshape(n, _L)
        o_ref[...] = (jnp.dot(mean2, wl_ref[...],
                              preferred_element_type=jnp.float32)
                      + jnp.dot(h_ref[pl.ds(0, n), :], wr_ref[...],
                                preferred_element_type=jnp.float32)
                      + b2_ref[...])
    return body


def _make_segsum(n_pad, nblk_tile, mode):
    mesh = plsc.VectorSubcoreMesh(core_axis_name="c", subcore_axis_name="s")
    rps = n_pad // 16
    nrd = n_pad // 16
    nrd_blk = nrd // _BLK
    drps = nrd // 16
    grp = 8
    grows = grp * _BLK
    assert nblk_tile % (2 * grp) == 0 and nrd % _BLK == 0 and drps % 8 == 0
    ngrp = nblk_tile // grp

    out_type = [jax.ShapeDtypeStruct((2, n_pad, _L), jnp.float32)]
    scratch = [
        pltpu.VMEM((nblk_tile, _BLK), jnp.int32),
        pltpu.VMEM((nblk_tile, _BLK), jnp.int32),
        pltpu.VMEM((2, grows, _L), jnp.float32),
        pltpu.VMEM_SHARED((n_pad, _L), jnp.float32),
        pltpu.VMEM_SHARED((n_pad, _L), jnp.float32),
        pltpu.SemaphoreType.DMA((2,)),
        pltpu.SemaphoreType.DMA((2,)),
    ]
    if mode == "deg":
        out_type.append(jax.ShapeDtypeStruct((2, nrd, _L), jnp.float32))
        scratch += [
            pltpu.VMEM((nrd_blk, _BLK), jnp.int32),
            pltpu.VMEM((nrd, _L), jnp.float32),
            pltpu.VMEM_SHARED((nrd, _L), jnp.float32),
            pltpu.SemaphoreType.DMA,
        ]
    else:
        out_type.append(jax.ShapeDtypeStruct((n_pad, _L), jnp.float32))
        scratch += [
            pltpu.VMEM((rps, _L), jnp.float32),
            pltpu.VMEM((rps, _L), jnp.float32),
            pltpu.VMEM((rps, _L), jnp.float32),
            pltpu.VMEM((2, drps, _L), jnp.float32),
            pltpu.VMEM((1, _L), jnp.float32),
            pltpu.VMEM((rps, _L), jnp.float32),
        ]

    def body(y_hbm, src_hbm, dst_hbm, zc_hbm, *rest):
        if mode == "deg":
            (id_hbm, out_hbm, degout_hbm, src_v, dst_v, rows_v, acc_sh,
             y_sh, sem_g, sem_s, id_v, deg_v, dacc_sh, sem_d) = rest
        else:
            (a_hbm, dg_hbm, b1_hbm, out_hbm, h_hbm, src_v, dst_v, rows_v,
             acc_sh, y_sh, sem_g, sem_s, ap_v, bp_v, z1_v, dg_v, b1_v,
             h_v) = rest

        c = lax.axis_index("c")
        s = lax.axis_index("s")
        wid = s * 2 + c

        def issue_gathers(k, buf):
            for b in range(grp):
                pltpu.async_copy(
                    y_sh.at[src_v.at[k * grp + b]],
                    rows_v.at[buf, pl.ds(b * _BLK, _BLK)], sem_g.at[buf])

        def drain(sem):
            pltpu.make_async_copy(
                y_hbm.at[pl.ds(0, grows)], rows_v.at[0], sem).wait()

        start = wid * nblk_tile
        my_rows = pl.ds(s * rps, rps)
        pltpu.sync_copy(src_hbm.at[pl.ds(start, nblk_tile)], src_v)
        pltpu.sync_copy(dst_hbm.at[pl.ds(start, nblk_tile)], dst_v)
        pltpu.sync_copy(zc_hbm.at[my_rows], acc_sh.at[my_rows])
        if mode == "deg":
            my_drows = pl.ds(s * drps, drps)
            pltpu.sync_copy(y_hbm.at[my_rows], y_sh.at[my_rows])
            pltpu.sync_copy(id_hbm, id_v)
            pltpu.sync_copy(zc_hbm.at[pl.ds(0, nrd)], deg_v)
            pltpu.sync_copy(zc_hbm.at[pl.ds(0, drps)], dacc_sh.at[my_drows])
        else:
            my_drows = pl.ds(s * drps, drps)
            pltpu.sync_copy(a_hbm.at[0, my_rows], ap_v)
            pltpu.sync_copy(a_hbm.at[1, my_rows], bp_v)
            pltpu.sync_copy(y_hbm.at[my_rows], z1_v)
            pltpu.sync_copy(dg_hbm.at[0, my_drows], dg_v.at[0])
            pltpu.sync_copy(dg_hbm.at[1, my_drows], dg_v.at[1])
            pltpu.sync_copy(b1_hbm, b1_v)
            b1vec = b1_v[0, :]

            @pl.loop(0, drps)
            def _(r):
                deg16 = jnp.maximum(dg_v[0, r, :] + dg_v[1, r, :], 1.0)
                for i in range(_L):
                    row = r * _L + i
                    dsc = jnp.sum(jnp.where(
                        lax.iota(jnp.int32, _L) == i, deg16, 0.0))
                    dvec = jnp.full((_L,), dsc, jnp.float32)
                    mean = (ap_v[row, :] + bp_v[row, :]) / dvec
                    h_v[row, :] = jnp.maximum(
                        mean + b1vec + z1_v[row, :], 0.0)

            pltpu.sync_copy(h_v, y_sh.at[my_rows])

            @pl.when(c == 0)
            def _():
                pltpu.sync_copy(h_v, h_hbm.at[my_rows])

        plsc.subcore_barrier()
        issue_gathers(0, 0)
        issue_gathers(1, 1)

        ones16 = jnp.ones((_L,), jnp.float32)

        @pl.loop(0, ngrp // 2)
        def _(g2):
            for buf in range(2):
                k = g2 * 2 + buf
                drain(sem_g.at[buf])
                for b in range(grp):
                    j = k * grp + b
                    pltpu.async_copy(
                        rows_v.at[buf, pl.ds(b * _BLK, _BLK)],
                        acc_sh.at[dst_v.at[j]], sem_s.at[buf], add=True)
                if mode == "deg":
                    for b in range(grp):
                        j = k * grp + b
                        for q in range(_BLK // _L):
                            d16 = dst_v.at[j][pl.ds(q * _L, _L)]
                            plsc.addupdate_scatter(
                                deg_v,
                                [lax.shift_right_logical(d16, 4),
                                 lax.bitwise_and(d16, 15)],
                                ones16)
                drain(sem_s.at[buf])

                @pl.when(k + 2 < ngrp)
                def _():
                    issue_gathers(k + 2, buf)

        if mode == "deg":
            for b in range(nrd_blk):
                pltpu.async_copy(deg_v.at[pl.ds(b * _BLK, _BLK)],
                                 dacc_sh.at[id_v.at[b]], sem_d, add=True)
            pltpu.make_async_copy(
                y_hbm.at[pl.ds(0, nrd)], deg_v, sem_d).wait()

        plsc.subcore_barrier()

        pltpu.sync_copy(acc_sh.at[my_rows], out_hbm.at[c, my_rows])
        if mode == "deg":
            my_drows = pl.ds(s * drps, drps)
            pltpu.sync_copy(dacc_sh.at[my_drows], degout_hbm.at[c, my_drows])

    return pl.kernel(
        body,
        out_type=tuple(out_type),
        mesh=mesh,
        scratch_types=scratch,
        compiler_params=pltpu.CompilerParams(use_tc_tiling_on_sc=False,
                                             needs_layout_passes=False),
    )


@jax.jit
def kernel(x, edge_index, W1_l, b1, W1_r, W2_l, b2, W2_r):
    n, d = x.shape
    h_dim = W1_l.shape[1]
    e = edge_index.shape[1]
    assert h_dim == _L and W2_l.shape[1] == _L

    blk_per_tile = -(-e // (_BLK * _NW))
    blk_per_tile = -(-blk_per_tile // 16) * 16
    e_pad = blk_per_tile * _BLK * _NW
    n_pad = -(-(n + 1) // 2048) * 2048
    assert e % _BLK == 0
    e_blk, eb_pad = e // _BLK, e_pad // _BLK

    segsum_deg = _make_segsum(n_pad, blk_per_tile, mode="deg")
    segsum_h = _make_segsum(n_pad, blk_per_tile, mode="h")

    y1p, z1p, src, dst, zc, idx_id = pl.pallas_call(
        _make_proj_body(n, n_pad, e_blk, eb_pad),
        out_shape=[jax.ShapeDtypeStruct((n_pad, _L), jnp.float32),
                   jax.ShapeDtypeStruct((n_pad, _L), jnp.float32),
                   jax.ShapeDtypeStruct((eb_pad, _BLK), jnp.int32),
                   jax.ShapeDtypeStruct((eb_pad, _BLK), jnp.int32),
                   jax.ShapeDtypeStruct((n_pad, _L), jnp.float32),
                   jax.ShapeDtypeStruct((n_pad // 16 // _BLK, _BLK),
                                        jnp.int32)],
    )(x, W1_l, W1_r, edge_index)

    agg1p, degp = segsum_deg(y1p, src, dst, zc, idx_id)

    agg2p, h = segsum_h(z1p, src, dst, zc, agg1p, degp, b1.reshape(1, _L))

    out = pl.pallas_call(
        _make_final_body(n),
        out_shape=jax.ShapeDtypeStruct((n, _L), jnp.float32),
    )(agg2p, degp, h, W2_l, W2_r, b2.reshape(1, _L))
    return out
